# baseline (speedup 1.0000x reference)
import functools

import jax
import jax.numpy as jnp
from jax import lax
from jax.experimental import pallas as pl
from jax.experimental.pallas import tpu as pltpu
from jax.experimental.pallas import tpu_sc as plsc

_NC = 2
_NS = 16


def _pad_to(n: int, m: int) -> int:
    return ((n + m - 1) // m) * m


def _make_deg_kernel(E: int, Np: int):
    K = 80
    ept = E // _NS
    n_chunks = ept // K
    assert ept % K == 0 and Np % (16 * _NS) == 0
    sl = Np // _NS
    mesh = plsc.VectorSubcoreMesh(core_axis_name="c", subcore_axis_name="s")

    @functools.partial(
        pl.kernel,
        out_type=jax.ShapeDtypeStruct((_NC * Np,), jnp.float32),
        mesh=mesh,
        scratch_types=[
            pltpu.VMEM((n_chunks, K), jnp.int32),
            pltpu.VMEM((K,), jnp.float32),
            pltpu.VMEM((Np // _NS,), jnp.float32),
            pltpu.VMEM_SHARED((Np,), jnp.float32),
        ],
    )
    def deg_kernel(ei_hbm, out_hbm, idx_v, ones_v, buf_v, acc_sh):
        c = lax.axis_index("c")
        s = lax.axis_index("s")
        for j in range(sl // 16):
            buf_v[pl.ds(j * 16, 16)] = jnp.zeros((16,), jnp.float32)
        pltpu.sync_copy(buf_v, acc_sh.at[pl.ds(s * sl, sl)])
        pltpu.sync_copy(ei_hbm.at[c, s], idx_v)
        for j in range(K // 16):
            ones_v[pl.ds(j * 16, 16)] = jnp.ones((16,), jnp.float32)
        plsc.subcore_barrier()

        def body(i, _):
            pltpu.sync_copy(ones_v, acc_sh.at[idx_v.at[i]], add=True)
            return ()

        lax.fori_loop(0, n_chunks, body, ())
        plsc.subcore_barrier()
        pltpu.sync_copy(acc_sh.at[pl.ds(s * sl, sl)], buf_v)
        pltpu.sync_copy(buf_v, out_hbm.at[pl.ds(c * Np + s * sl, sl)])

    return deg_kernel


_KC = 80


def _make_scatter_kernel(ept: int, Nn: int, H: int, K: int, nbuf: int):
    nw = _NC * _NS
    n_chunks = ept // K
    assert ept % K == 0 and n_chunks >= 2 * nbuf and K >= _KC
    Na = _pad_to(Nn + 1, _NS * _KC)
    rpt = Na // _NS
    zc = rpt // _KC
    mesh = plsc.VectorSubcoreMesh(core_axis_name="c", subcore_axis_name="s")

    _NBUF = nbuf
    tc_tiling = (H % 128 == 0)

    scratch = [
        pltpu.VMEM((ept,) if tc_tiling else (n_chunks, K), jnp.int32),
        pltpu.VMEM((n_chunks, K), jnp.int32),
    ] + [pltpu.VMEM((K, H), jnp.float32)] * _NBUF \
      + [pltpu.VMEM_SHARED((Na, H), jnp.float32)] \
      + [pltpu.SemaphoreType.DMA] * _NBUF

    def body(ei_rows_src, ei_cols_src, g_hbm, out_hbm, ridx_v, cidx_v, rest):
        rows = rest[:_NBUF]
        acc_sh = rest[_NBUF]
        sems = rest[_NBUF + 1:]
        c = lax.axis_index("c")
        s = lax.axis_index("s")
        for j in range(zc):
            pltpu.sync_copy(rows[0].at[pl.ds(0, _KC)],
                            acc_sh.at[pl.ds(s * rpt + j * _KC, _KC)])
        pltpu.sync_copy(ei_rows_src, ridx_v)
        pltpu.sync_copy(ei_cols_src, cidx_v)
        plsc.subcore_barrier()

        def rsl(i):
            if tc_tiling:
                return ridx_v.at[pl.ds(i * K, K)]
            return ridx_v.at[i]

        def start(i, b):
            pltpu.async_copy(g_hbm.at[rsl(i)], rows[b], sems[b])

        def wait(b):
            pltpu.make_async_copy(g_hbm.at[rsl(0)], rows[b], sems[b]).wait()

        def scat(i, b):
            pltpu.sync_copy(rows[b], acc_sh.at[cidx_v.at[i]], add=True)

        for b in range(_NBUF):
            start(b, b)
        main_iters = n_chunks // _NBUF - 1

        def loop_body(j, _):
            for b in range(_NBUF):
                i = j * _NBUF + b
                wait(b)
                scat(i, b)
                start(i + _NBUF, b)
            return ()

        lax.fori_loop(0, main_iters, loop_body, ())
        for i in range(main_iters * _NBUF, n_chunks):
            b = i % _NBUF
            wait(b)
            scat(i, b)
            if i + _NBUF < n_chunks:
                start(i + _NBUF, b)
        plsc.subcore_barrier()
        co = lax.axis_index("c")
        for j in range(zc):
            pltpu.sync_copy(acc_sh.at[pl.ds(s * rpt + j * _KC, _KC)],
                            rows[0].at[pl.ds(0, _KC)])
            pltpu.sync_copy(rows[0].at[pl.ds(0, _KC)],
                            out_hbm.at[co, pl.ds(s * rpt + j * _KC, _KC)])

    params = pltpu.CompilerParams(use_tc_tiling_on_sc=tc_tiling)
    kw = dict(out_type=jax.ShapeDtypeStruct((_NC, Na, H), jnp.float32),
              mesh=mesh, scratch_types=scratch, compiler_params=params)

    if tc_tiling:
        @functools.partial(pl.kernel, **kw)
        def scatter_tc(eir_hbm, eic_hbm, g_hbm, zeros_hbm, out_hbm,
                       ridx_v, cidx_v, *rest):
            c = lax.axis_index("c")
            s = lax.axis_index("s")
            wid = c * _NS + s
            pltpu.sync_copy(zeros_hbm, rest[0].at[pl.ds(0, _KC)])
            body(eir_hbm.at[0, wid], eic_hbm.at[1, wid], g_hbm, out_hbm,
                 ridx_v, cidx_v, rest)

        return scatter_tc

    @functools.partial(pl.kernel, **kw)
    def scatter_sc(ei_hbm, g_hbm, zeros_hbm, out_hbm,
                   ridx_v, cidx_v, *rest):
        c = lax.axis_index("c")
        s = lax.axis_index("s")
        wid = c * _NS + s
        pltpu.sync_copy(zeros_hbm, rest[0].at[pl.ds(0, _KC)])
        body(ei_hbm.at[0, wid], ei_hbm.at[1, wid], g_hbm, out_hbm,
             ridx_v, cidx_v, rest)

    return scatter_sc


_BN = 2000


def _matmul_bias(x, W, b):
    Nn, Din = x.shape
    Hh = W.shape[1]

    def body(x_ref, w_ref, b_ref, h_ref):
        h_ref[...] = jnp.dot(x_ref[...], w_ref[...],
                             preferred_element_type=jnp.float32) + b_ref[...]

    return pl.pallas_call(
        body,
        grid=(Nn // _BN,),
        in_specs=[
            pl.BlockSpec((_BN, Din), lambda i: (i, 0)),
            pl.BlockSpec((Din, Hh), lambda i: (0, 0)),
            pl.BlockSpec((1, Hh), lambda i: (0, 0)),
        ],
        out_specs=pl.BlockSpec((_BN, Hh), lambda i: (i, 0)),
        out_shape=jax.ShapeDtypeStruct((Nn, Hh), jnp.float32),
    )(x, W, b.reshape(1, -1))


def _scale(h, dis):
    Nn, Hh = h.shape

    def body(h_ref, d_ref, g_ref):
        g_ref[...] = d_ref[...] * h_ref[...]

    return pl.pallas_call(
        body,
        grid=(Nn // _BN,),
        in_specs=[
            pl.BlockSpec((_BN, Hh), lambda i: (i, 0)),
            pl.BlockSpec((_BN, 1), lambda i: (i, 0)),
        ],
        out_specs=pl.BlockSpec((_BN, Hh), lambda i: (i, 0)),
        out_shape=jax.ShapeDtypeStruct((Nn, Hh), jnp.float32),
    )(h, dis)


def _combine_relu_mm_scale(h, Sp, coef, dis, W, b):
    Nn, H1 = h.shape
    H2 = W.shape[1]

    def body(h_ref, sp_ref, c_ref, d_ref, w_ref, b_ref, h2_ref, g2_ref):
        S = sp_ref[0, :, :H1] + sp_ref[1, :, :H1]
        z = c_ref[...] * h_ref[...] + d_ref[...] * S
        a = jnp.maximum(z, 0.0)
        h2 = jnp.dot(a, w_ref[...],
                     preferred_element_type=jnp.float32) + b_ref[...]
        h2_ref[...] = h2
        g2_ref[...] = d_ref[...] * h2

    return pl.pallas_call(
        body,
        grid=(Nn // _BN,),
        in_specs=[
            pl.BlockSpec((_BN, H1), lambda i: (i, 0)),
            pl.BlockSpec((_NC, _BN, H1), lambda i: (0, i, 0)),
            pl.BlockSpec((_BN, 1), lambda i: (i, 0)),
            pl.BlockSpec((_BN, 1), lambda i: (i, 0)),
            pl.BlockSpec((H1, H2), lambda i: (0, 0)),
            pl.BlockSpec((1, H2), lambda i: (0, 0)),
        ],
        out_specs=[pl.BlockSpec((_BN, H2), lambda i: (i, 0))] * 2,
        out_shape=[jax.ShapeDtypeStruct((Nn, H2), jnp.float32)] * 2,
    )(h, Sp, coef, dis, W, b.reshape(1, -1))


def _combine_logsoftmax(h, Sp, coef, dis):
    Nn, H2 = h.shape

    def body(h_ref, sp_ref, c_ref, d_ref, o_ref):
        S = sp_ref[0] + sp_ref[1]
        z = c_ref[...] * h_ref[...] + d_ref[...] * S
        m = jnp.max(z, axis=1, keepdims=True)
        e = jnp.exp(z - m)
        lse = jnp.log(jnp.sum(e, axis=1, keepdims=True)) + m
        o_ref[...] = z - lse

    return pl.pallas_call(
        body,
        grid=(Nn // _BN,),
        in_specs=[
            pl.BlockSpec((_BN, H2), lambda i: (i, 0)),
            pl.BlockSpec((_NC, _BN, H2), lambda i: (0, i, 0)),
            pl.BlockSpec((_BN, 1), lambda i: (i, 0)),
            pl.BlockSpec((_BN, 1), lambda i: (i, 0)),
        ],
        out_specs=pl.BlockSpec((_BN, H2), lambda i: (i, 0)),
        out_shape=jax.ShapeDtypeStruct((Nn, H2), jnp.float32),
    )(h, Sp, coef, dis)


def kernel(x, edge_index, W1, b1, W2, b2):
    Nn, D = x.shape
    H = W1.shape[1]
    E = edge_index.shape[1]
    Np = _pad_to(Nn, 16 * _NS)

    nw = _NC * _NS
    ept = E // nw
    K1, NB1 = 128, 4
    K2, NB2 = 96, 2
    ep1 = _pad_to(ept, K1)
    ep2 = _pad_to(ept, K2)
    ei_deg = edge_index.reshape(2, _NS, (E // _NS) // 80, 80)
    ei3 = edge_index.reshape(2, nw, ept)

    def _pad_edges(ep):
        r = jnp.pad(ei3[0], ((0, 0), (0, ep - ept)))
        cc = jnp.pad(ei3[1], ((0, 0), (0, ep - ept)), constant_values=Nn)
        return jnp.stack([r, cc])

    ei_p1 = _pad_edges(ep1)
    ei_col1 = ei_p1.reshape(2, nw, ep1 // K1, K1)
    ei_p2 = _pad_edges(ep2)
    ei_col2 = ei_p2.reshape(2, nw, ep2 // K2, K2)
    zeros_s1 = jnp.zeros((_KC, H), jnp.float32)
    zeros_s2 = jnp.zeros((_KC, D), jnp.float32)

    cnt = _make_deg_kernel(E, Np)(ei_deg).reshape(_NC, Np)
    cnt_row = cnt[0, :Nn]
    cnt_col = cnt[1, :Nn]
    dis1 = lax.rsqrt(cnt_row + 1.0)
    dis = dis1.reshape(Nn, 1)
    coef = (cnt_col + 1.0 + dis1 * dis1).reshape(Nn, 1)

    h1 = _matmul_bias(x, W1, b1)
    g1 = _scale(h1, dis)
    S1p = _make_scatter_kernel(ep1, Nn, H, K1, NB1)(ei_col1, g1, zeros_s1)
    h2, g2 = _combine_relu_mm_scale(h1, S1p, coef, dis, W2, b2)
    S2p = _make_scatter_kernel(ep2, Nn, D, K2, NB2)(ei_p2, ei_col2, g2,
                                                    zeros_s2)
    return _combine_logsoftmax(h2, S2p, coef, dis)


# speedup vs baseline: 1.5812x; 1.5812x over previous
import functools

import jax
import jax.numpy as jnp
from jax import lax
from jax.experimental import pallas as pl
from jax.experimental.pallas import tpu as pltpu
from jax.experimental.pallas import tpu_sc as plsc

_NC = 2
_NS = 16


def _pad_to(n: int, m: int) -> int:
    return ((n + m - 1) // m) * m


def _make_deg_kernel(E: int, Np: int):
    K = 80
    ept = E // _NS
    n_chunks = ept // K
    assert ept % K == 0 and Np % (16 * _NS) == 0
    sl = Np // _NS
    mesh = plsc.VectorSubcoreMesh(core_axis_name="c", subcore_axis_name="s")

    @functools.partial(
        pl.kernel,
        out_type=jax.ShapeDtypeStruct((_NC * Np,), jnp.float32),
        mesh=mesh,
        scratch_types=[
            pltpu.VMEM((n_chunks, K), jnp.int32),
            pltpu.VMEM((K,), jnp.float32),
            pltpu.VMEM((Np // _NS,), jnp.float32),
            pltpu.VMEM_SHARED((Np,), jnp.float32),
        ],
    )
    def deg_kernel(ei_hbm, out_hbm, idx_v, ones_v, buf_v, acc_sh):
        c = lax.axis_index("c")
        s = lax.axis_index("s")
        for j in range(sl // 16):
            buf_v[pl.ds(j * 16, 16)] = jnp.zeros((16,), jnp.float32)
        pltpu.sync_copy(buf_v, acc_sh.at[pl.ds(s * sl, sl)])
        pltpu.sync_copy(ei_hbm.at[c, s], idx_v)
        for j in range(K // 16):
            ones_v[pl.ds(j * 16, 16)] = jnp.ones((16,), jnp.float32)
        plsc.subcore_barrier()

        def body(i, _):
            pltpu.sync_copy(ones_v, acc_sh.at[idx_v.at[i]], add=True)
            return ()

        lax.fori_loop(0, n_chunks, body, ())
        plsc.subcore_barrier()
        pltpu.sync_copy(acc_sh.at[pl.ds(s * sl, sl)], buf_v)
        pltpu.sync_copy(buf_v, out_hbm.at[pl.ds(c * Np + s * sl, sl)])

    return deg_kernel


_KS = 80
_NBUF = 2


def _make_scatter_kernel(E: int, Nn: int, H: int, nbuf: int = _NBUF):
    _NBUF = nbuf
    nw = _NC * _NS
    ept = E // nw
    K = _KS
    n_chunks = ept // K
    assert ept % K == 0 and n_chunks >= 2 * _NBUF
    Na = _pad_to(Nn, _NS * K)
    rpt = Na // _NS
    zc = rpt // K
    mesh = plsc.VectorSubcoreMesh(core_axis_name="c", subcore_axis_name="s")

    tc_tiling = (H % 128 == 0)

    scratch = [
        pltpu.VMEM((ept,) if tc_tiling else (n_chunks, K), jnp.int32),
        pltpu.VMEM((n_chunks, K), jnp.int32),
    ] + [pltpu.VMEM((K, H), jnp.float32)] * _NBUF \
      + [pltpu.VMEM_SHARED((Na, H), jnp.float32)] \
      + [pltpu.SemaphoreType.DMA] * _NBUF

    def body(ei_rows_src, ei_cols_src, g_hbm, out_hbm, ridx_v, cidx_v, rest):
        rows = rest[:_NBUF]
        acc_sh = rest[_NBUF]
        sems = rest[_NBUF + 1:]
        c = lax.axis_index("c")
        s = lax.axis_index("s")
        for j in range(zc):
            pltpu.sync_copy(rows[0], acc_sh.at[pl.ds(s * rpt + j * K, K)])
        pltpu.sync_copy(ei_rows_src, ridx_v)
        pltpu.sync_copy(ei_cols_src, cidx_v)
        plsc.subcore_barrier()

        def rsl(i):
            if tc_tiling:
                return ridx_v.at[pl.ds(i * K, K)]
            return ridx_v.at[i]

        def start(i, b):
            pltpu.async_copy(g_hbm.at[rsl(i)], rows[b], sems[b])

        def wait(b):
            pltpu.make_async_copy(g_hbm.at[rsl(0)], rows[b], sems[b]).wait()

        def scat(i, b):
            pltpu.sync_copy(rows[b], acc_sh.at[cidx_v.at[i]], add=True)

        for b in range(_NBUF):
            start(b, b)
        main_iters = n_chunks // _NBUF - 1

        def loop_body(j, _):
            for b in range(_NBUF):
                i = j * _NBUF + b
                wait(b)
                scat(i, b)
                start(i + _NBUF, b)
            return ()

        lax.fori_loop(0, main_iters, loop_body, ())
        for i in range(main_iters * _NBUF, n_chunks):
            b = i % _NBUF
            wait(b)
            scat(i, b)
            if i + _NBUF < n_chunks:
                start(i + _NBUF, b)
        plsc.subcore_barrier()
        co = lax.axis_index("c")
        for j in range(zc):
            pltpu.sync_copy(acc_sh.at[pl.ds(s * rpt + j * K, K)], rows[0])
            pltpu.sync_copy(rows[0],
                            out_hbm.at[co, pl.ds(s * rpt + j * K, K)])

    params = pltpu.CompilerParams(use_tc_tiling_on_sc=tc_tiling)
    kw = dict(out_type=jax.ShapeDtypeStruct((_NC, Na, H), jnp.float32),
              mesh=mesh, scratch_types=scratch, compiler_params=params)

    if tc_tiling:
        @functools.partial(pl.kernel, **kw)
        def scatter_tc(eir_hbm, eic_hbm, g_hbm, zeros_hbm, out_hbm,
                       ridx_v, cidx_v, *rest):
            c = lax.axis_index("c")
            s = lax.axis_index("s")
            wid = c * _NS + s
            pltpu.sync_copy(zeros_hbm, rest[0])
            body(eir_hbm.at[0, wid], eic_hbm.at[1, wid], g_hbm, out_hbm,
                 ridx_v, cidx_v, rest)

        return scatter_tc

    @functools.partial(pl.kernel, **kw)
    def scatter_sc(ei_hbm, g_hbm, zeros_hbm, out_hbm,
                   ridx_v, cidx_v, *rest):
        c = lax.axis_index("c")
        s = lax.axis_index("s")
        wid = c * _NS + s
        pltpu.sync_copy(zeros_hbm, rest[0])
        body(ei_hbm.at[0, wid], ei_hbm.at[1, wid], g_hbm, out_hbm,
             ridx_v, cidx_v, rest)

    return scatter_sc


_BN = 2000


def _matmul_bias(x, W, b):
    Nn, Din = x.shape
    Hh = W.shape[1]

    def body(x_ref, w_ref, b_ref, h_ref):
        h_ref[...] = jnp.dot(x_ref[...], w_ref[...],
                             preferred_element_type=jnp.float32) + b_ref[...]

    return pl.pallas_call(
        body,
        grid=(Nn // _BN,),
        in_specs=[
            pl.BlockSpec((_BN, Din), lambda i: (i, 0)),
            pl.BlockSpec((Din, Hh), lambda i: (0, 0)),
            pl.BlockSpec((1, Hh), lambda i: (0, 0)),
        ],
        out_specs=pl.BlockSpec((_BN, Hh), lambda i: (i, 0)),
        out_shape=jax.ShapeDtypeStruct((Nn, Hh), jnp.float32),
    )(x, W, b.reshape(1, -1))


def _scale(h, dis):
    Nn, Hh = h.shape

    def body(h_ref, d_ref, g_ref):
        g_ref[...] = d_ref[...] * h_ref[...]

    return pl.pallas_call(
        body,
        grid=(Nn // _BN,),
        in_specs=[
            pl.BlockSpec((_BN, Hh), lambda i: (i, 0)),
            pl.BlockSpec((_BN, 1), lambda i: (i, 0)),
        ],
        out_specs=pl.BlockSpec((_BN, Hh), lambda i: (i, 0)),
        out_shape=jax.ShapeDtypeStruct((Nn, Hh), jnp.float32),
    )(h, dis)


def _combine_relu_mm_scale(h, Sp, coef, dis, W, b):
    Nn, H1 = h.shape
    H2 = W.shape[1]

    def body(h_ref, sp_ref, c_ref, d_ref, w_ref, b_ref, h2_ref, g2_ref):
        S = sp_ref[0, :, :H1] + sp_ref[1, :, :H1]
        z = c_ref[...] * h_ref[...] + d_ref[...] * S
        a = jnp.maximum(z, 0.0)
        h2 = jnp.dot(a, w_ref[...],
                     preferred_element_type=jnp.float32) + b_ref[...]
        h2_ref[...] = h2
        g2_ref[...] = d_ref[...] * h2

    return pl.pallas_call(
        body,
        grid=(Nn // _BN,),
        in_specs=[
            pl.BlockSpec((_BN, H1), lambda i: (i, 0)),
            pl.BlockSpec((_NC, _BN, H1), lambda i: (0, i, 0)),
            pl.BlockSpec((_BN, 1), lambda i: (i, 0)),
            pl.BlockSpec((_BN, 1), lambda i: (i, 0)),
            pl.BlockSpec((H1, H2), lambda i: (0, 0)),
            pl.BlockSpec((1, H2), lambda i: (0, 0)),
        ],
        out_specs=[pl.BlockSpec((_BN, H2), lambda i: (i, 0))] * 2,
        out_shape=[jax.ShapeDtypeStruct((Nn, H2), jnp.float32)] * 2,
    )(h, Sp, coef, dis, W, b.reshape(1, -1))


def _combine_logsoftmax(h, Sp, coef, dis):
    Nn, H2 = h.shape

    def body(h_ref, sp_ref, c_ref, d_ref, o_ref):
        S = sp_ref[0] + sp_ref[1]
        z = c_ref[...] * h_ref[...] + d_ref[...] * S
        m = jnp.max(z, axis=1, keepdims=True)
        e = jnp.exp(z - m)
        lse = jnp.log(jnp.sum(e, axis=1, keepdims=True)) + m
        o_ref[...] = z - lse

    return pl.pallas_call(
        body,
        grid=(Nn // _BN,),
        in_specs=[
            pl.BlockSpec((_BN, H2), lambda i: (i, 0)),
            pl.BlockSpec((_NC, _BN, H2), lambda i: (0, i, 0)),
            pl.BlockSpec((_BN, 1), lambda i: (i, 0)),
            pl.BlockSpec((_BN, 1), lambda i: (i, 0)),
        ],
        out_specs=pl.BlockSpec((_BN, H2), lambda i: (i, 0)),
        out_shape=jax.ShapeDtypeStruct((Nn, H2), jnp.float32),
    )(h, Sp, coef, dis)


def kernel(x, edge_index, W1, b1, W2, b2):
    Nn, D = x.shape
    H = W1.shape[1]
    E = edge_index.shape[1]
    Np = _pad_to(Nn, 16 * _NS)

    nw = _NC * _NS
    ei_deg = edge_index.reshape(2, _NS, (E // _NS) // 80, 80)
    ei_row = edge_index.reshape(2, nw, E // nw)
    ei_col = edge_index.reshape(2, nw, (E // nw) // _KS, _KS)
    zeros_s1 = jnp.zeros((_KS, H), jnp.float32)
    zeros_s2 = jnp.zeros((_KS, D), jnp.float32)

    cnt = _make_deg_kernel(E, Np)(ei_deg).reshape(_NC, Np)
    cnt_row = cnt[0, :Nn]
    cnt_col = cnt[1, :Nn]
    dis1 = lax.rsqrt(cnt_row + 1.0)
    dis = dis1.reshape(Nn, 1)
    coef = (cnt_col + 1.0 + dis1 * dis1).reshape(Nn, 1)

    h1 = _matmul_bias(x, W1, b1)
    g1 = _scale(h1, dis)
    S1p = _make_scatter_kernel(E, Nn, H, nbuf=4)(ei_col, g1, zeros_s1)
    h2, g2 = _combine_relu_mm_scale(h1, S1p, coef, dis, W2, b2)
    S2p = _make_scatter_kernel(E, Nn, D)(ei_row, ei_col, g2, zeros_s2)
    return _combine_logsoftmax(h2, S2p, coef, dis)


# speedup vs baseline: 1.6818x; 1.0636x over previous
import functools

import jax
import jax.numpy as jnp
from jax import lax
from jax.experimental import pallas as pl
from jax.experimental.pallas import tpu as pltpu
from jax.experimental.pallas import tpu_sc as plsc

_NC = 2
_NS = 16


def _pad_to(n: int, m: int) -> int:
    return ((n + m - 1) // m) * m


def _make_deg_kernel(E: int, Np: int):
    K = 80
    ept = E // _NS
    n_chunks = ept // K
    assert ept % K == 0 and Np % (16 * _NS) == 0
    sl = Np // _NS
    mesh = plsc.VectorSubcoreMesh(core_axis_name="c", subcore_axis_name="s")

    @functools.partial(
        pl.kernel,
        out_type=jax.ShapeDtypeStruct((_NC * Np,), jnp.float32),
        mesh=mesh,
        scratch_types=[
            pltpu.VMEM((n_chunks, K), jnp.int32),
            pltpu.VMEM((K,), jnp.float32),
            pltpu.VMEM((Np // _NS,), jnp.float32),
            pltpu.VMEM_SHARED((Np,), jnp.float32),
            pltpu.SemaphoreType.DMA,
        ],
    )
    def deg_kernel(ei_hbm, out_hbm, idx_v, ones_v, buf_v, acc_sh, sem):
        c = lax.axis_index("c")
        s = lax.axis_index("s")
        for j in range(sl // 16):
            buf_v[pl.ds(j * 16, 16)] = jnp.zeros((16,), jnp.float32)
        pltpu.sync_copy(buf_v, acc_sh.at[pl.ds(s * sl, sl)])
        pltpu.sync_copy(ei_hbm.at[c, s], idx_v)
        for j in range(K // 16):
            ones_v[pl.ds(j * 16, 16)] = jnp.ones((16,), jnp.float32)
        plsc.subcore_barrier()

        def body(i, _):
            pltpu.async_copy(ones_v, acc_sh.at[idx_v.at[i]], sem, add=True)
            return ()

        lax.fori_loop(0, n_chunks, body, ())

        def drain(i, _):
            pltpu.make_async_copy(ones_v, acc_sh.at[idx_v.at[0]], sem).wait()
            return ()

        lax.fori_loop(0, n_chunks, drain, ())
        plsc.subcore_barrier()
        pltpu.sync_copy(acc_sh.at[pl.ds(s * sl, sl)], buf_v)
        pltpu.sync_copy(buf_v, out_hbm.at[pl.ds(c * Np + s * sl, sl)])

    return deg_kernel


_KS = 80
_NBUF = 2


def _make_scatter_kernel(E: int, Nn: int, H: int, nbuf: int = _NBUF):
    _NBUF = nbuf
    nw = _NC * _NS
    ept = E // nw
    K = _KS
    n_chunks = ept // K
    assert ept % K == 0 and n_chunks >= 2 * _NBUF
    Na = _pad_to(Nn, _NS * K)
    rpt = Na // _NS
    zc = rpt // K
    mesh = plsc.VectorSubcoreMesh(core_axis_name="c", subcore_axis_name="s")

    tc_tiling = (H % 128 == 0)

    scratch = [
        pltpu.VMEM((ept,) if tc_tiling else (n_chunks, K), jnp.int32),
        pltpu.VMEM((n_chunks, K), jnp.int32),
    ] + [pltpu.VMEM((K, H), jnp.float32)] * _NBUF \
      + [pltpu.VMEM_SHARED((Na, H), jnp.float32)] \
      + [pltpu.SemaphoreType.DMA] * _NBUF

    def body(ei_rows_src, ei_cols_src, g_hbm, out_hbm, ridx_v, cidx_v, rest):
        rows = rest[:_NBUF]
        acc_sh = rest[_NBUF]
        sems = rest[_NBUF + 1:]
        c = lax.axis_index("c")
        s = lax.axis_index("s")
        for j in range(zc):
            pltpu.sync_copy(rows[0], acc_sh.at[pl.ds(s * rpt + j * K, K)])
        pltpu.sync_copy(ei_rows_src, ridx_v)
        pltpu.sync_copy(ei_cols_src, cidx_v)
        plsc.subcore_barrier()

        def rsl(i):
            if tc_tiling:
                return ridx_v.at[pl.ds(i * K, K)]
            return ridx_v.at[i]

        def start(i, b):
            pltpu.async_copy(g_hbm.at[rsl(i)], rows[b], sems[b])

        def wait(b):
            pltpu.make_async_copy(g_hbm.at[rsl(0)], rows[b], sems[b]).wait()

        def scat(i, b):
            pltpu.sync_copy(rows[b], acc_sh.at[cidx_v.at[i]], add=True)

        for b in range(_NBUF):
            start(b, b)
        main_iters = n_chunks // _NBUF - 1

        def loop_body(j, _):
            for b in range(_NBUF):
                i = j * _NBUF + b
                wait(b)
                scat(i, b)
                start(i + _NBUF, b)
            return ()

        lax.fori_loop(0, main_iters, loop_body, ())
        for i in range(main_iters * _NBUF, n_chunks):
            b = i % _NBUF
            wait(b)
            scat(i, b)
            if i + _NBUF < n_chunks:
                start(i + _NBUF, b)
        plsc.subcore_barrier()
        co = lax.axis_index("c")
        for j in range(zc):
            pltpu.sync_copy(acc_sh.at[pl.ds(s * rpt + j * K, K)], rows[0])
            pltpu.sync_copy(rows[0],
                            out_hbm.at[co, pl.ds(s * rpt + j * K, K)])

    params = pltpu.CompilerParams(use_tc_tiling_on_sc=tc_tiling)
    kw = dict(out_type=jax.ShapeDtypeStruct((_NC, Na, H), jnp.float32),
              mesh=mesh, scratch_types=scratch, compiler_params=params)

    if tc_tiling:
        @functools.partial(pl.kernel, **kw)
        def scatter_tc(eir_hbm, eic_hbm, g_hbm, zeros_hbm, out_hbm,
                       ridx_v, cidx_v, *rest):
            c = lax.axis_index("c")
            s = lax.axis_index("s")
            wid = c * _NS + s
            pltpu.sync_copy(zeros_hbm, rest[0])
            body(eir_hbm.at[0, wid], eic_hbm.at[1, wid], g_hbm, out_hbm,
                 ridx_v, cidx_v, rest)

        return scatter_tc

    @functools.partial(pl.kernel, **kw)
    def scatter_sc(ei_hbm, g_hbm, zeros_hbm, out_hbm,
                   ridx_v, cidx_v, *rest):
        c = lax.axis_index("c")
        s = lax.axis_index("s")
        wid = c * _NS + s
        pltpu.sync_copy(zeros_hbm, rest[0])
        body(ei_hbm.at[0, wid], ei_hbm.at[1, wid], g_hbm, out_hbm,
             ridx_v, cidx_v, rest)

    return scatter_sc


_BN = 2000


def _matmul_bias(x, W, b):
    Nn, Din = x.shape
    Hh = W.shape[1]

    def body(x_ref, w_ref, b_ref, h_ref):
        h_ref[...] = jnp.dot(x_ref[...], w_ref[...],
                             preferred_element_type=jnp.float32) + b_ref[...]

    return pl.pallas_call(
        body,
        grid=(Nn // _BN,),
        in_specs=[
            pl.BlockSpec((_BN, Din), lambda i: (i, 0)),
            pl.BlockSpec((Din, Hh), lambda i: (0, 0)),
            pl.BlockSpec((1, Hh), lambda i: (0, 0)),
        ],
        out_specs=pl.BlockSpec((_BN, Hh), lambda i: (i, 0)),
        out_shape=jax.ShapeDtypeStruct((Nn, Hh), jnp.float32),
    )(x, W, b.reshape(1, -1))


def _scale(h, dis):
    Nn, Hh = h.shape

    def body(h_ref, d_ref, g_ref):
        g_ref[...] = d_ref[...] * h_ref[...]

    return pl.pallas_call(
        body,
        grid=(Nn // _BN,),
        in_specs=[
            pl.BlockSpec((_BN, Hh), lambda i: (i, 0)),
            pl.BlockSpec((_BN, 1), lambda i: (i, 0)),
        ],
        out_specs=pl.BlockSpec((_BN, Hh), lambda i: (i, 0)),
        out_shape=jax.ShapeDtypeStruct((Nn, Hh), jnp.float32),
    )(h, dis)


def _combine_relu_mm_scale(h, Sp, coef, dis, W, b):
    Nn, H1 = h.shape
    H2 = W.shape[1]

    def body(h_ref, sp_ref, c_ref, d_ref, w_ref, b_ref, h2_ref, g2_ref):
        S = sp_ref[0, :, :H1] + sp_ref[1, :, :H1]
        z = c_ref[...] * h_ref[...] + d_ref[...] * S
        a = jnp.maximum(z, 0.0)
        h2 = jnp.dot(a, w_ref[...],
                     preferred_element_type=jnp.float32) + b_ref[...]
        h2_ref[...] = h2
        g2_ref[...] = d_ref[...] * h2

    return pl.pallas_call(
        body,
        grid=(Nn // _BN,),
        in_specs=[
            pl.BlockSpec((_BN, H1), lambda i: (i, 0)),
            pl.BlockSpec((_NC, _BN, H1), lambda i: (0, i, 0)),
            pl.BlockSpec((_BN, 1), lambda i: (i, 0)),
            pl.BlockSpec((_BN, 1), lambda i: (i, 0)),
            pl.BlockSpec((H1, H2), lambda i: (0, 0)),
            pl.BlockSpec((1, H2), lambda i: (0, 0)),
        ],
        out_specs=[pl.BlockSpec((_BN, H2), lambda i: (i, 0))] * 2,
        out_shape=[jax.ShapeDtypeStruct((Nn, H2), jnp.float32)] * 2,
    )(h, Sp, coef, dis, W, b.reshape(1, -1))


def _combine_logsoftmax(h, Sp, coef, dis):
    Nn, H2 = h.shape

    def body(h_ref, sp_ref, c_ref, d_ref, o_ref):
        S = sp_ref[0] + sp_ref[1]
        z = c_ref[...] * h_ref[...] + d_ref[...] * S
        m = jnp.max(z, axis=1, keepdims=True)
        e = jnp.exp(z - m)
        lse = jnp.log(jnp.sum(e, axis=1, keepdims=True)) + m
        o_ref[...] = z - lse

    return pl.pallas_call(
        body,
        grid=(Nn // _BN,),
        in_specs=[
            pl.BlockSpec((_BN, H2), lambda i: (i, 0)),
            pl.BlockSpec((_NC, _BN, H2), lambda i: (0, i, 0)),
            pl.BlockSpec((_BN, 1), lambda i: (i, 0)),
            pl.BlockSpec((_BN, 1), lambda i: (i, 0)),
        ],
        out_specs=pl.BlockSpec((_BN, H2), lambda i: (i, 0)),
        out_shape=jax.ShapeDtypeStruct((Nn, H2), jnp.float32),
    )(h, Sp, coef, dis)


def kernel(x, edge_index, W1, b1, W2, b2):
    Nn, D = x.shape
    H = W1.shape[1]
    E = edge_index.shape[1]
    Np = _pad_to(Nn, 16 * _NS)

    nw = _NC * _NS
    ei_deg = edge_index.reshape(2, _NS, (E // _NS) // 80, 80)
    ei_row = edge_index.reshape(2, nw, E // nw)
    ei_col = edge_index.reshape(2, nw, (E // nw) // _KS, _KS)
    zeros_s1 = jnp.zeros((_KS, H), jnp.float32)
    zeros_s2 = jnp.zeros((_KS, D), jnp.float32)

    cnt = _make_deg_kernel(E, Np)(ei_deg).reshape(_NC, Np)
    cnt_row = cnt[0, :Nn]
    cnt_col = cnt[1, :Nn]
    dis1 = lax.rsqrt(cnt_row + 1.0)
    dis = dis1.reshape(Nn, 1)
    coef = (cnt_col + 1.0 + dis1 * dis1).reshape(Nn, 1)

    h1 = _matmul_bias(x, W1, b1)
    g1 = _scale(h1, dis)
    S1p = _make_scatter_kernel(E, Nn, H, nbuf=8)(ei_col, g1, zeros_s1)
    h2, g2 = _combine_relu_mm_scale(h1, S1p, coef, dis, W2, b2)
    S2p = _make_scatter_kernel(E, Nn, D)(ei_row, ei_col, g2, zeros_s2)
    return _combine_logsoftmax(h2, S2p, coef, dis)
